# Initial kernel scaffold; baseline (speedup 1.0000x reference)
import jax
import jax.numpy as jnp
from jax.experimental import pallas as pl


def kernel(inputs, W):
    raise NotImplementedError("write your pallas kernel here")


# speedup vs baseline: 1.1715x; 1.1715x over previous
import functools

import jax
import jax.numpy as jnp
from jax import lax
from jax.experimental import pallas as pl
from jax.experimental.pallas import tpu as pltpu
from jax.experimental.pallas import tpu_sc as plsc

_NUM_EMBEDDINGS = 8192
_EMBEDDING_DIM = 32
_COMMITMENT_COST = 0.25
_N_TOKENS = 16384
_TILE_M = 256
_GRID = _N_TOKENS // _TILE_M

_NC = 2
_NS = 16
_NW = _NC * _NS
_B_PER_W = _N_TOKENS // _NW


def _argmin_body(x_ref, w_ref, idx_ref, msum_ref):
    i = pl.program_id(0)
    x = x_ref[...]
    w = w_ref[...]
    mm = lax.dot_general(x, w, (((1,), (1,)), ((), ())),
                         preferred_element_type=jnp.float32)
    xsq = jnp.sum(x * x, axis=1, keepdims=True)
    wsq = jnp.sum(w * w, axis=1)[None, :]
    dist = (xsq + wsq) - 2.0 * mm
    minv = jnp.min(dist, axis=1, keepdims=True)
    ids = lax.broadcasted_iota(jnp.int32, dist.shape, 1)
    idx = jnp.min(jnp.where(dist == minv, ids, _NUM_EMBEDDINGS), axis=1)
    idx_ref[...] = idx.astype(jnp.int32)[:, None]

    @pl.when(i == 0)
    def _():
        msum_ref[...] = jnp.zeros_like(msum_ref)

    msum_ref[...] = msum_ref[...] + jnp.sum(minv)


def _tc_argmin(flat, w):
    return pl.pallas_call(
        _argmin_body,
        grid=(_GRID,),
        in_specs=[
            pl.BlockSpec((_TILE_M, _EMBEDDING_DIM), lambda i: (i, 0)),
            pl.BlockSpec((_NUM_EMBEDDINGS, _EMBEDDING_DIM), lambda i: (0, 0)),
        ],
        out_specs=[
            pl.BlockSpec((_TILE_M, 1), lambda i: (i, 0)),
            pl.BlockSpec((1, 1), lambda i: (0, 0)),
        ],
        out_shape=[
            jax.ShapeDtypeStruct((_N_TOKENS, 1), jnp.int32),
            jax.ShapeDtypeStruct((1, 1), jnp.float32),
        ],
    )(flat, w)


@functools.cache
def _sc_gather_fn():
    mesh = plsc.VectorSubcoreMesh(core_axis_name="c", subcore_axis_name="s")

    @functools.partial(
        pl.kernel,
        mesh=mesh,
        out_type=jax.ShapeDtypeStruct((_N_TOKENS, _EMBEDDING_DIM),
                                      jnp.float32),
        scratch_types=[
            pltpu.VMEM((_B_PER_W,), jnp.int32),
            pltpu.VMEM((_B_PER_W, _EMBEDDING_DIM), jnp.float32),
            pltpu.SemaphoreType.DMA,
        ],
        compiler_params=pltpu.CompilerParams(use_tc_tiling_on_sc=False),
    )
    def _gather(table_hbm, idx_hbm, out_hbm, idx_v, rows_v, sem):
        wid = lax.axis_index("s") * _NC + lax.axis_index("c")
        base = wid * _B_PER_W
        pltpu.sync_copy(idx_hbm.at[pl.ds(base, _B_PER_W)], idx_v)
        pltpu.async_copy(table_hbm.at[idx_v], rows_v, sem).wait()
        pltpu.sync_copy(rows_v, out_hbm.at[pl.ds(base, _B_PER_W)])

    return _gather


def kernel(inputs, W):
    x = jnp.transpose(inputs, (0, 2, 3, 1))
    B, H, Wd, D = x.shape
    flat = x.reshape(-1, D)
    idx2, msum = _tc_argmin(flat, W)
    idx = idx2.reshape(-1)
    q_flat = _sc_gather_fn()(W, idx)
    loss = (1.0 + _COMMITMENT_COST) * msum[0, 0] / (flat.shape[0] * D)
    quantized = q_flat.reshape(x.shape)
    quant_nchw = jnp.transpose(quantized, (0, 3, 1, 2))
    return loss, quant_nchw, idx.reshape(B, H, Wd)
